# baseline (speedup 1.0000x reference)
import functools

import jax
import jax.numpy as jnp
from jax import lax
from jax.experimental import pallas as pl
from jax.experimental.pallas import tpu as pltpu
from jax.experimental.pallas import tpu_sc as plsc

NUM_ACTIONS = 1000
EMBED_DIM = 64
BATCH = 16384
HIST = 50

NC = 2
NS = 16
NW = NC * NS
LANES = 16
VPR = EMBED_DIM // LANES

N_FLAT = BATCH * HIST
PER_W = N_FLAT // NW
CHUNK = 128
N_CHUNKS = PER_W // CHUNK
GROUPS = CHUNK // LANES
NBUF = 4
CHUNK_ELEMS = CHUNK * EMBED_DIM


def _make_kernel():
    mesh = plsc.VectorSubcoreMesh(
        core_axis_name="c", subcore_axis_name="s", num_cores=NC, num_subcores=NS
    )

    @functools.partial(
        pl.kernel,
        out_type=jax.ShapeDtypeStruct((N_FLAT * EMBED_DIM,), jnp.float32),
        mesh=mesh,
        scratch_types=[
            pltpu.VMEM((NUM_ACTIONS * EMBED_DIM,), jnp.float32),
            pltpu.VMEM((PER_W,), jnp.int32),
            pltpu.VMEM((NBUF, CHUNK_ELEMS), jnp.float32),
            pltpu.SemaphoreType.DMA((NBUF,)),
        ],
        compiler_params=pltpu.CompilerParams(
            use_tc_tiling_on_sc=False, needs_layout_passes=False
        ),
    )
    def gather_kernel(idx_hbm, table_hbm, out_hbm, table_v, idx_v, rows_v, osem):
        wid = lax.axis_index("s") * NC + lax.axis_index("c")
        base = wid * PER_W
        pltpu.sync_copy(table_hbm, table_v)
        pltpu.sync_copy(idx_hbm.at[wid], idx_v)

        def wait_write(j, b):
            pltpu.make_async_copy(
                rows_v.at[b],
                out_hbm.at[pl.ds((base + j * CHUNK) * EMBED_DIM, CHUNK_ELEMS)],
                osem.at[b],
            ).wait()

        def body(s, carry):
            for b in range(NBUF):
                j = s * NBUF + b

                @pl.when(j >= NBUF)
                def _(j=j, b=b):
                    wait_write(j - NBUF, b)

                buf = rows_v.at[b]

                @plsc.parallel_loop(0, GROUPS, unroll=4)
                def group(g):
                    idxv = idx_v[pl.ds(j * CHUNK + g * LANES, LANES)]
                    rs = [idxv[l] * EMBED_DIM for l in range(LANES)]
                    for l in range(LANES):
                        w = g * (LANES * EMBED_DIM) + l * EMBED_DIM
                        for k in range(VPR):
                            buf[pl.ds(w + k * LANES, LANES)] = table_v[
                                pl.ds(rs[l] + k * LANES, LANES)
                            ]

                pltpu.async_copy(
                    buf,
                    out_hbm.at[pl.ds((base + j * CHUNK) * EMBED_DIM, CHUNK_ELEMS)],
                    osem.at[b],
                )
            return carry

        lax.fori_loop(0, N_CHUNKS // NBUF, body, 0)
        for b in range(NBUF):
            wait_write(N_CHUNKS - NBUF + b, b)

    return gather_kernel


_gather = _make_kernel()


@jax.jit
def kernel(action_indices, embedding_table):
    idx = action_indices.astype(jnp.int32).reshape(NW, PER_W)
    out = _gather(idx, embedding_table.reshape(-1))
    return out.reshape(BATCH, HIST, EMBED_DIM)


# speedup vs baseline: 1.1560x; 1.1560x over previous
import functools

import jax
import jax.numpy as jnp
from jax import lax
from jax.experimental import pallas as pl
from jax.experimental.pallas import tpu as pltpu
from jax.experimental.pallas import tpu_sc as plsc

NUM_ACTIONS = 1000
EMBED_DIM = 64
BATCH = 16384
HIST = 50

NC = 2
NS = 16
NW = NC * NS
LANES = 16
VPR = EMBED_DIM // LANES

N_FLAT = BATCH * HIST
PER_W = N_FLAT // NW
CHUNK = 128
N_CHUNKS = PER_W // CHUNK
GROUPS = CHUNK // LANES
NBUF = 4
STREAM_BUFS = (0, 1)
COMPUTE_BUFS = (2, 3)


def _make_kernel():
    mesh = plsc.VectorSubcoreMesh(
        core_axis_name="c", subcore_axis_name="s", num_cores=NC, num_subcores=NS
    )

    @functools.partial(
        pl.kernel,
        out_type=jax.ShapeDtypeStruct((N_FLAT, EMBED_DIM), jnp.float32),
        mesh=mesh,
        scratch_types=[
            pltpu.VMEM_SHARED((NUM_ACTIONS, EMBED_DIM), jnp.float32),
            pltpu.VMEM((NUM_ACTIONS, EMBED_DIM), jnp.float32),
            pltpu.VMEM((N_CHUNKS, CHUNK), jnp.int32),
            pltpu.VMEM((NBUF, CHUNK, EMBED_DIM), jnp.float32),
            pltpu.SemaphoreType.DMA((NBUF,)),
            pltpu.SemaphoreType.DMA((NBUF,)),
        ],
        compiler_params=pltpu.CompilerParams(
            use_tc_tiling_on_sc=False, needs_layout_passes=False
        ),
    )
    def gather_kernel(
        idx_hbm, table_hbm, out_hbm, table_s, table_v, idx_v, rows_v, gsem, osem
    ):
        sid = lax.axis_index("s")
        wid = sid * NC + lax.axis_index("c")
        base = wid * PER_W

        @pl.when(sid == 0)
        def _():
            pltpu.sync_copy(table_hbm, table_s)

        pltpu.sync_copy(table_hbm, table_v)
        pltpu.sync_copy(idx_hbm.at[wid], idx_v)
        plsc.subcore_barrier()

        def wait_gather(j, b):
            pltpu.make_async_copy(
                table_s.at[idx_v.at[j]], rows_v.at[b], gsem.at[b]
            ).wait()

        def wait_write(j, b):
            pltpu.make_async_copy(
                rows_v.at[b], out_hbm.at[pl.ds(base + j * CHUNK, CHUNK)], osem.at[b]
            ).wait()

        def start_write(j, b):
            pltpu.async_copy(
                rows_v.at[b], out_hbm.at[pl.ds(base + j * CHUNK, CHUNK)], osem.at[b]
            )

        def body(g, carry):
            for b in STREAM_BUFS:
                j = g * NBUF + b

                @pl.when(j >= NBUF)
                def _(j=j, b=b):
                    wait_write(j - NBUF, b)

                pltpu.async_copy(
                    table_s.at[idx_v.at[j]], rows_v.at[b], gsem.at[b]
                )

            for b in COMPUTE_BUFS:
                j = g * NBUF + b

                @pl.when(j >= NBUF)
                def _(j=j, b=b):
                    wait_write(j - NBUF, b)

                buf = rows_v.at[b]

                @plsc.parallel_loop(0, GROUPS, unroll=2)
                def grp(gg, j=j, buf=buf):
                    idxv = idx_v[j, pl.ds(gg * LANES, LANES)]
                    for l in range(LANES):
                        r = idxv[l]
                        row = gg * LANES + l
                        for k in range(VPR):
                            buf[row, pl.ds(k * LANES, LANES)] = table_v[
                                r, pl.ds(k * LANES, LANES)
                            ]

                start_write(j, b)

            for b in STREAM_BUFS:
                j = g * NBUF + b
                wait_gather(j, b)
                start_write(j, b)

            return carry

        lax.fori_loop(0, N_CHUNKS // NBUF, body, 0)
        for b in range(NBUF):
            wait_write(N_CHUNKS - NBUF + b, b)

    return gather_kernel


_gather = _make_kernel()


@jax.jit
def kernel(action_indices, embedding_table):
    idx = action_indices.astype(jnp.int32).reshape(NW, N_CHUNKS, CHUNK)
    out = _gather(idx, embedding_table)
    return out.reshape(BATCH, HIST, EMBED_DIM)
